# Initial kernel scaffold; baseline (speedup 1.0000x reference)
import jax
import jax.numpy as jnp
from jax.experimental import pallas as pl


def kernel(edge_index_list, num_nodes_list, perturb_one_hot, emb_table, W1, b1, gamma, beta, W2, b2):
    raise NotImplementedError("write your pallas kernel here")


# speedup vs baseline: 8.6274x; 8.6274x over previous
import functools

import jax
import jax.numpy as jnp
import numpy as np
from jax import lax
from jax.experimental import pallas as pl
from jax.experimental.pallas import tpu as pltpu
from jax.experimental.pallas import tpu_sc as plsc

MAX_HOP = 4
EMBED = 64
HIDDEN = 256
OUT = 128
N = 50000
E = 800000
B = 2

NC = 2
NS = 16
L = 16

HALF = N // 2
HWORDS = HALF * EMBED
CHUNK = 2000
EPT = E // NS
NCHUNK = EPT // CHUNK
ROWS16 = CHUNK // L
BROWS = 782
BWORDS = BROWS * EMBED
LASTW = (HALF - 31 * BROWS) * EMBED
ZCH = BWORDS // 4

_SQRT_HALF = 0.7071067811865476


_AW = 512
_ANB = (N + _AW - 1) // _AW


def _argmax_body(p_ref, o_ref):
    o_ref[0, 0] = jnp.argmax(p_ref[...], axis=0).astype(jnp.int32)


_argmax_call = pl.pallas_call(
    _argmax_body,
    grid=(_ANB,),
    in_specs=[pl.BlockSpec((NCELLS := 64, _AW), lambda i: (0, i))],
    out_specs=pl.BlockSpec((1, 1, _AW), lambda i: (i, 0, 0)),
    out_shape=jax.ShapeDtypeStruct((_ANB, 1, _AW), jnp.int32),
)


def _sc_hist_body(edges_hbm, idx_hbm, c_hbm,
                  row_v, col_v, cell_v, band0_v, band1_v, idxst_v,
                  idx_sh):
    cid = lax.axis_index("c")
    sid = lax.axis_index("s")

    @pl.when(sid < 10)
    def _():
        pltpu.sync_copy(idx_hbm.at[pl.ds(sid * 5000, 5000)], idxst_v)
        pltpu.sync_copy(idxst_v, idx_sh.at[pl.ds(sid * 5000, 5000)])
    plsc.subcore_barrier()

    cbase = cid * HWORDS
    base0 = cbase + sid * BWORDS
    base1 = cbase + (16 + sid) * BWORDS
    ones = jnp.full((L,), 1.0, jnp.float32)

    for g in range(B):
        zv = jnp.zeros((L,), jnp.float32)

        def _zero(j, _):
            band0_v[pl.ds(j * L, L)] = zv
            band1_v[pl.ds(j * L, L)] = zv
            return 0
        lax.fori_loop(0, BWORDS // L, _zero, 0)

        def _chunk(ch, _):
            pltpu.sync_copy(edges_hbm.at[g, 0, sid, ch], row_v)
            pltpu.sync_copy(edges_hbm.at[g, 1, sid, ch], col_v)
            pltpu.sync_copy(idx_sh.at[col_v], cell_v)

            def _lanes(j, _):
                sl = pl.ds(j * L, L)
                t = row_v[sl] * EMBED + cell_v[sl]
                r0 = t - base0
                ok0 = (r0 >= 0) & (r0 < BWORDS)
                plsc.addupdate_scatter(
                    band0_v, [jnp.where(ok0, r0, BWORDS)], ones)
                r1 = t - base1
                ok1 = (r1 >= 0) & (r1 < BWORDS)
                plsc.addupdate_scatter(
                    band1_v, [jnp.where(ok1, r1, BWORDS)], ones)
                return 0
            lax.fori_loop(0, ROWS16, _lanes, 0)
            return 0
        lax.fori_loop(0, NCHUNK, _chunk, 0)

        gbase = g * N * EMBED
        pltpu.sync_copy(band0_v.at[pl.ds(0, BWORDS)],
                        c_hbm.at[pl.ds(gbase + base0, BWORDS)])

        @pl.when(sid < 15)
        def _():
            pltpu.sync_copy(band1_v.at[pl.ds(0, BWORDS)],
                            c_hbm.at[pl.ds(gbase + base1, BWORDS)])

        @pl.when(sid == 15)
        def _():
            pltpu.sync_copy(band1_v.at[pl.ds(0, LASTW)],
                            c_hbm.at[pl.ds(gbase + base1, LASTW)])


@functools.cache
def _sc_hist_call():
    mesh = plsc.VectorSubcoreMesh(core_axis_name="c", subcore_axis_name="s")
    return pl.kernel(
        _sc_hist_body,
        mesh=mesh,
        out_type=jax.ShapeDtypeStruct((B * N * EMBED,), jnp.float32),
        compiler_params=pltpu.CompilerParams(needs_layout_passes=False),
        scratch_types=[
            pltpu.VMEM((CHUNK,), jnp.int32),
            pltpu.VMEM((CHUNK,), jnp.int32),
            pltpu.VMEM((CHUNK,), jnp.int32),
            pltpu.VMEM((BWORDS + L,), jnp.float32),
            pltpu.VMEM((BWORDS + L,), jnp.float32),
            pltpu.VMEM((5000,), jnp.int32),
            pltpu.VMEM_SHARED((N,), jnp.int32),
        ],
    )


def _prep_body(e_ref, s_ref, w1_ref, m_ref):
    rep = jnp.dot(e_ref[...], s_ref[...],
                  preferred_element_type=jnp.float32,
                  precision=lax.Precision.HIGHEST)
    m_ref[...] = jnp.dot(rep, w1_ref[...],
                         preferred_element_type=jnp.float32,
                         precision=lax.Precision.HIGHEST)


_prep_call = pl.pallas_call(
    _prep_body,
    out_shape=jax.ShapeDtypeStruct((EMBED, HIDDEN), jnp.float32),
)

_S_REP = np.kron(np.eye(EMBED, dtype=np.float32),
                 np.ones((1, MAX_HOP), dtype=np.float32))


_R = 1000
_RNB = N // _R


def _mlp_body(c_ref, m_ref, p_ref, w2_ref, o_ref):
    h = jnp.dot(c_ref[0], m_ref[...],
                preferred_element_type=jnp.float32,
                precision=lax.Precision.HIGHEST) + p_ref[0]
    mu = jnp.mean(h, axis=-1, keepdims=True)
    var = jnp.mean((h - mu) ** 2, axis=-1, keepdims=True)
    h = (h - mu) * lax.rsqrt(var + 1e-5) * p_ref[1] + p_ref[2]
    h = h * 0.5 * (1.0 + lax.erf(h * _SQRT_HALF))
    o_ref[0] = jnp.dot(h, w2_ref[...],
                       preferred_element_type=jnp.float32,
                       precision=lax.Precision.HIGHEST) + p_ref[3, :OUT]


_mlp_call = pl.pallas_call(
    _mlp_body,
    grid=(B, _RNB),
    in_specs=[
        pl.BlockSpec((1, _R, EMBED), lambda g, i: (g, i, 0)),
        pl.BlockSpec((EMBED, HIDDEN), lambda g, i: (0, 0)),
        pl.BlockSpec((8, HIDDEN), lambda g, i: (0, 0)),
        pl.BlockSpec((HIDDEN, OUT), lambda g, i: (0, 0)),
    ],
    out_specs=pl.BlockSpec((1, _R, OUT), lambda g, i: (g, i, 0)),
    out_shape=jax.ShapeDtypeStruct((B, N, OUT), jnp.float32),
)


def kernel(edge_index_list, num_nodes_list, perturb_one_hot, emb_table,
           W1, b1, gamma, beta, W2, b2):
    del num_nodes_list

    idx = _argmax_call(perturb_one_hot).reshape(-1)[:N]

    edges5 = edge_index_list.reshape(B, 2, NS, NCHUNK, CHUNK)
    c = _sc_hist_call()(edges5, idx).reshape(B, N, EMBED)

    m = _prep_call(emb_table, _S_REP, W1)
    params = jnp.zeros((8, HIDDEN), jnp.float32)
    params = params.at[0].set(b1).at[1].set(gamma).at[2].set(beta)
    params = params.at[3, :OUT].set(b2)

    return _mlp_call(c, m, params, W2)
